# Initial kernel scaffold; baseline (speedup 1.0000x reference)
import functools

import jax
import jax.numpy as jnp
from jax import lax
from jax.experimental import pallas as pl
from jax.experimental.pallas import tpu as pltpu
from jax.experimental.pallas import tpu_sc as plsc


def _build_sc_kernel(S, F, num_cores, num_subcores, L):
    NW = num_cores * num_subcores
    B = 128
    SPW = S // NW
    NCHUNK = SPW // B
    ROWS_PER_CHUNK = B * 8 // 128

    mesh = plsc.VectorSubcoreMesh(core_axis_name="c", subcore_axis_name="s")

    @functools.partial(
        pl.kernel,
        mesh=mesh,
        out_type=jax.ShapeDtypeStruct((S, F), jnp.float32),
        scratch_types=[
            pltpu.VMEM((ROWS_PER_CHUNK, 128), jnp.int32),
            pltpu.VMEM((B, 3), jnp.float32),
            pltpu.VMEM((B, 3), jnp.float32),
            pltpu.VMEM((B, 3), jnp.float32),
            pltpu.VMEM((B * 8, F), jnp.float32),
            pltpu.VMEM((B, F), jnp.float32),
            pltpu.SemaphoreType.DMA,
            pltpu.SemaphoreType.DMA,
        ],
    )
    def sc_interp(xyz_hbm, c0_hbm, c7_hbm, idx_hbm, field_hbm, out_hbm,
                  idx_v, xyz_v, c0_v, c7_v, rows_v, out_v, sem_meta, sem_rows):
        wid = lax.axis_index("s") * num_cores + lax.axis_index("c")
        w0 = wid * SPW
        iota = lax.iota(jnp.int32, L)
        d0 = jnp.zeros((L,), jnp.int32)
        d1 = jnp.full((L,), 1, jnp.int32)
        d2 = jnp.full((L,), 2, jnp.int32)

        def chunk_body(c, carry):
            base = w0 + c * B
            rowb = wid * (SPW * 8 // 128) + c * ROWS_PER_CHUNK
            cps = [
                pltpu.async_copy(idx_hbm.at[pl.ds(rowb, ROWS_PER_CHUNK)], idx_v, sem_meta),
                pltpu.async_copy(xyz_hbm.at[pl.ds(base, B)], xyz_v, sem_meta),
                pltpu.async_copy(c0_hbm.at[pl.ds(base, B)], c0_v, sem_meta),
                pltpu.async_copy(c7_hbm.at[pl.ds(base, B)], c7_v, sem_meta),
            ]
            for cp in cps:
                cp.wait()
            gs = [
                pltpu.async_copy(field_hbm.at[idx_v.at[j]],
                                 rows_v.at[pl.ds(j * 128, 128)], sem_rows)
                for j in range(ROWS_PER_CHUNK)
            ]
            for gp in gs:
                gp.wait()

            for g in range(B // L):
                svec = iota + g * L
                gx = plsc.load_gather(xyz_v, [svec, d0])
                gy = plsc.load_gather(xyz_v, [svec, d1])
                gz = plsc.load_gather(xyz_v, [svec, d2])
                a0x = plsc.load_gather(c0_v, [svec, d0])
                a0y = plsc.load_gather(c0_v, [svec, d1])
                a0z = plsc.load_gather(c0_v, [svec, d2])
                a7x = plsc.load_gather(c7_v, [svec, d0])
                a7y = plsc.load_gather(c7_v, [svec, d1])
                a7z = plsc.load_gather(c7_v, [svec, d2])
                cx = (gx - a0x) / (a7x - a0x)
                cy = (gy - a0y) / (a7y - a0y)
                cz = (gz - a0z) / (a7z - a0z)
                fx = (1.0 - cx, cx)
                fy = (1.0 - cy, cy)
                fz = (1.0 - cz, cz)
                wts = [fx[(k >> 2) & 1] * fy[(k >> 1) & 1] * fz[k & 1]
                       for k in range(8)]
                rvecs = [svec * 8 + k for k in range(8)]

                def f_body(f, fcarry):
                    col = jnp.full((L,), f, jnp.int32)
                    acc = wts[0] * plsc.load_gather(rows_v, [rvecs[0], col])
                    for k in range(1, 8):
                        acc = acc + wts[k] * plsc.load_gather(rows_v, [rvecs[k], col])
                    plsc.store_scatter(out_v, [svec, col], acc)
                    return fcarry

                lax.fori_loop(0, F, f_body, None)

            pltpu.sync_copy(out_v, out_hbm.at[pl.ds(base, B)])
            return carry

        lax.fori_loop(0, NCHUNK, chunk_body, None)

    return sc_interp


def kernel(xyz, cell_xyz, vertices_id, field):
    S = xyz.shape[0]
    P, F = field.shape
    cell0 = cell_xyz[:, 0, :]
    cell7 = cell_xyz[:, 7, :]
    idx2d = vertices_id.astype(jnp.int32).reshape(-1, 128)
    info = plsc.get_sparse_core_info()
    sc_interp = _build_sc_kernel(S, F, info.num_cores, info.num_subcores,
                                 info.num_lanes)
    return sc_interp(xyz, cell0, cell7, idx2d, field)


# speedup vs baseline: 1.5501x; 1.5501x over previous
import functools

import jax
import jax.numpy as jnp
from jax import lax
from jax.experimental import pallas as pl
from jax.experimental.pallas import tpu as pltpu
from jax.experimental.pallas import tpu_sc as plsc


def _build_sc_kernel(S, F, num_cores, num_subcores, L):
    NW = num_cores * num_subcores
    B = 128
    SPW = S // NW
    NCHUNK = SPW // B
    ROWS_PER_CHUNK = B * 8 // 128

    mesh = plsc.VectorSubcoreMesh(core_axis_name="c", subcore_axis_name="s")

    @functools.partial(
        pl.kernel,
        mesh=mesh,
        compiler_params=pltpu.CompilerParams(
            needs_layout_passes=False, use_tc_tiling_on_sc=False),
        out_type=jax.ShapeDtypeStruct((S, F), jnp.float32),
        scratch_types=[
            pltpu.VMEM((ROWS_PER_CHUNK, 128), jnp.int32),
            pltpu.VMEM((9, B), jnp.float32),
            pltpu.VMEM((B * 8, F), jnp.float32),
            pltpu.VMEM((B, F), jnp.float32),
            pltpu.SemaphoreType.DMA,
            pltpu.SemaphoreType.DMA,
        ],
    )
    def sc_interp(meta_hbm, idx_hbm, field_hbm, out_hbm,
                  idx_v, meta_v, rows_v, out_v, sem_meta, sem_rows):
        wid = lax.axis_index("s") * num_cores + lax.axis_index("c")
        w0 = wid * SPW
        iota = lax.iota(jnp.int32, L)

        def chunk_body(c, carry):
            base = w0 + c * B
            rowb = wid * (SPW * 8 // 128) + c * ROWS_PER_CHUNK
            cps = [
                pltpu.async_copy(idx_hbm.at[pl.ds(rowb, ROWS_PER_CHUNK)], idx_v, sem_meta),
                pltpu.async_copy(meta_hbm.at[:, pl.ds(base, B)], meta_v, sem_meta),
            ]
            for cp in cps:
                cp.wait()
            gs = [
                pltpu.async_copy(field_hbm.at[idx_v.at[j]],
                                 rows_v.at[pl.ds(j * 128, 128)], sem_rows)
                for j in range(ROWS_PER_CHUNK)
            ]
            for gp in gs:
                gp.wait()

            for g in range(B // L):
                svec = iota + g * L
                sl = pl.ds(g * L, L)
                gx = meta_v[0, sl]
                gy = meta_v[1, sl]
                gz = meta_v[2, sl]
                a0x = meta_v[3, sl]
                a0y = meta_v[4, sl]
                a0z = meta_v[5, sl]
                a7x = meta_v[6, sl]
                a7y = meta_v[7, sl]
                a7z = meta_v[8, sl]
                cx = (gx - a0x) / (a7x - a0x)
                cy = (gy - a0y) / (a7y - a0y)
                cz = (gz - a0z) / (a7z - a0z)
                fx = (1.0 - cx, cx)
                fy = (1.0 - cy, cy)
                fz = (1.0 - cz, cz)
                wts = [fx[(k >> 2) & 1] * fy[(k >> 1) & 1] * fz[k & 1]
                       for k in range(8)]
                rvecs = [svec * 8 + k for k in range(8)]

                def f_body(f, fcarry):
                    col = jnp.full((L,), f, jnp.int32)
                    acc = wts[0] * plsc.load_gather(rows_v, [rvecs[0], col])
                    for k in range(1, 8):
                        acc = acc + wts[k] * plsc.load_gather(rows_v, [rvecs[k], col])
                    plsc.store_scatter(out_v, [svec, col], acc)
                    return fcarry

                lax.fori_loop(0, F, f_body, None)

            pltpu.sync_copy(out_v, out_hbm.at[pl.ds(base, B)])
            return carry

        lax.fori_loop(0, NCHUNK, chunk_body, None)

    return sc_interp


def kernel(xyz, cell_xyz, vertices_id, field):
    S = xyz.shape[0]
    P, F = field.shape
    meta = jnp.concatenate(
        [xyz.T, cell_xyz[:, 0, :].T, cell_xyz[:, 7, :].T], axis=0)
    idx2d = vertices_id.astype(jnp.int32).reshape(-1, 128)
    info = plsc.get_sparse_core_info()
    sc_interp = _build_sc_kernel(S, F, info.num_cores, info.num_subcores,
                                 info.num_lanes)
    return sc_interp(meta, idx2d, field)


# speedup vs baseline: 1.7585x; 1.1344x over previous
import functools

import jax
import jax.numpy as jnp
from jax import lax
from jax.experimental import pallas as pl
from jax.experimental.pallas import tpu as pltpu
from jax.experimental.pallas import tpu_sc as plsc


def _build_sc_kernel(S, F, num_cores, num_subcores, L):
    NW = num_cores * num_subcores
    B = 128
    SPW = S // NW
    NCHUNK = SPW // B
    IDXROWS = B * 8 // 128

    mesh = plsc.VectorSubcoreMesh(core_axis_name="c", subcore_axis_name="s")

    @functools.partial(
        pl.kernel,
        mesh=mesh,
        compiler_params=pltpu.CompilerParams(
            needs_layout_passes=False, use_tc_tiling_on_sc=False),
        out_type=jax.ShapeDtypeStruct((S, F), jnp.float32),
        scratch_types=[
            pltpu.VMEM((2, IDXROWS, 128), jnp.int32),
            pltpu.VMEM((2, 9, B), jnp.float32),
            pltpu.VMEM((2, B * 8, F), jnp.float32),
            pltpu.VMEM((8, B), jnp.float32),
            pltpu.VMEM((B, F), jnp.float32),
            pltpu.SemaphoreType.DMA,
            pltpu.SemaphoreType.DMA,
            pltpu.SemaphoreType.DMA,
            pltpu.SemaphoreType.DMA,
        ],
    )
    def sc_interp(meta_hbm, idx_hbm, field_hbm, out_hbm,
                  idx_v, meta_v, rows_v, wts_v, out_v,
                  semM0, semM1, semR0, semR1):
        wid = lax.axis_index("s") * num_cores + lax.axis_index("c")
        w0 = wid * SPW
        idxrow0 = wid * (SPW * 8 // 128)
        iota = lax.iota(jnp.int32, L)
        semM = (semM0, semM1)
        semR = (semR0, semR1)

        def meta_copies(c, p):
            base = w0 + c * B
            rowb = idxrow0 + c * IDXROWS
            return (
                pltpu.make_async_copy(idx_hbm.at[pl.ds(rowb, IDXROWS)],
                                      idx_v.at[p], semM[p]),
                pltpu.make_async_copy(meta_hbm.at[:, pl.ds(base, B)],
                                      meta_v.at[p], semM[p]),
            )

        def rows_copies(p):
            return tuple(
                pltpu.make_async_copy(field_hbm.at[idx_v.at[p].at[j]],
                                      rows_v.at[p].at[pl.ds(j * 128, 128)],
                                      semR[p])
                for j in range(IDXROWS)
            )

        def fire(cps):
            for cp in cps:
                cp.start()

        def drain(cps):
            for cp in cps:
                cp.wait()

        def phase_a(p):
            @plsc.parallel_loop(0, B // L)
            def g_body_a(g):
                sl = pl.ds(g * L, L)
                gx = meta_v[p, 0, sl]
                gy = meta_v[p, 1, sl]
                gz = meta_v[p, 2, sl]
                a0x = meta_v[p, 3, sl]
                a0y = meta_v[p, 4, sl]
                a0z = meta_v[p, 5, sl]
                a7x = meta_v[p, 6, sl]
                a7y = meta_v[p, 7, sl]
                a7z = meta_v[p, 8, sl]
                cx = (gx - a0x) / (a7x - a0x)
                cy = (gy - a0y) / (a7y - a0y)
                cz = (gz - a0z) / (a7z - a0z)
                fx = (1.0 - cx, cx)
                fy = (1.0 - cy, cy)
                fz = (1.0 - cz, cz)
                for k in range(8):
                    wts_v[k, sl] = (fx[(k >> 2) & 1] * fy[(k >> 1) & 1]
                                    * fz[k & 1])

        def phase_b(c, p):
            base = w0 + c * B
            rows2d = rows_v.at[p]

            @plsc.parallel_loop(0, B // L)
            def g_body_b(g):
                sl = pl.ds(g * L, L)
                svec = iota + g * L
                wts = [wts_v[k, sl] for k in range(8)]
                rb = svec * 8
                rvecs = [rb + k for k in range(8)]

                @plsc.parallel_loop(0, F // 2, unroll=4)
                def f_body(fi):
                    col0 = jnp.full((L,), fi * 2, jnp.int32)
                    col1 = col0 + 1
                    t0 = [wts[k] * plsc.load_gather(rows2d, [rvecs[k], col0])
                          for k in range(8)]
                    t1 = [wts[k] * plsc.load_gather(rows2d, [rvecs[k], col1])
                          for k in range(8)]
                    acc0 = (((t0[0] + t0[1]) + (t0[2] + t0[3]))
                            + ((t0[4] + t0[5]) + (t0[6] + t0[7])))
                    acc1 = (((t1[0] + t1[1]) + (t1[2] + t1[3]))
                            + ((t1[4] + t1[5]) + (t1[6] + t1[7])))
                    plsc.store_scatter(out_v, [svec, col0], acc0)
                    plsc.store_scatter(out_v, [svec, col1], acc1)
            pltpu.sync_copy(out_v, out_hbm.at[pl.ds(base, B)])

        def process(c, p, have_next, have_prefetch):
            phase_a(p)
            if have_next:
                drain(meta_copies(c + 1, 1 - p))
                fire(rows_copies(1 - p))
            drain(rows_copies(p))
            if have_prefetch:
                fire(meta_copies(c + 2, p))
            phase_b(c, p)

        fire(meta_copies(0, 0))
        drain(meta_copies(0, 0))
        fire(rows_copies(0))
        fire(meta_copies(1, 1))

        def pair_body(i, carry):
            c = i * 2
            process(c, 0, True, True)
            process(c + 1, 1, True, True)
            return carry

        lax.fori_loop(0, NCHUNK // 2 - 1, pair_body, None)
        process(NCHUNK - 2, 0, True, False)
        process(NCHUNK - 1, 1, False, False)

    return sc_interp


def kernel(xyz, cell_xyz, vertices_id, field):
    S = xyz.shape[0]
    P, F = field.shape
    meta = jnp.concatenate(
        [xyz.T, cell_xyz[:, 0, :].T, cell_xyz[:, 7, :].T], axis=0)
    idx2d = vertices_id.astype(jnp.int32).reshape(-1, 128)
    info = plsc.get_sparse_core_info()
    sc_interp = _build_sc_kernel(S, F, info.num_cores, info.num_subcores,
                                 info.num_lanes)
    return sc_interp(meta, idx2d, field)
